# baseline (speedup 1.0000x reference)
import functools
import math

import jax
import jax.numpy as jnp
from jax import lax
from jax.experimental import pallas as pl
from jax.experimental.pallas import tpu as pltpu
from jax.experimental.pallas import tpu_sc as plsc

EMB = 16
NFEAT = 26
BATCH = 4096
VOCAB = 163
MULT = 1000000.0

NC, NS, L = 2, 16, 16
NW = NC * NS
BPW = BATCH // NW
CHUNK = 16
NCHUNK = BPW // CHUNK
OUTW = NFEAT * 2 * EMB
SLICE = VOCAB * EMB
NQ = 5
NP9 = 7
QUAD_WORDS = NFEAT * NQ * 16 * EMB
T9_ROWS = NP9 * 9
T9_WORDS = NFEAT * T9_ROWS * EMB


def _sc_body(x_hbm, table_hbm, out_hbm,
             x_v, stage0, stage1, quad_v, t9_v, out0, out1,
             sx, s0, s1, so0, so1):
    wid = lax.axis_index("s") * NC + lax.axis_index("c")

    hx = pltpu.async_copy(
        x_hbm.at[pl.ds(wid * (NFEAT * BPW), NFEAT * BPW)], x_v, sx)
    h0 = pltpu.async_copy(table_hbm.at[pl.ds(0, SLICE)], stage0, s0)
    h1 = pltpu.async_copy(table_hbm.at[pl.ds(SLICE, SLICE)], stage1, s1)

    lane = lax.iota(jnp.int32, L)

    def build_feature(n, stage):
        def row(r):
            return plsc.load_gather(stage, [r * EMB + lane])

        const2 = row(2 * 20)
        for p in range(21, 32):
            const2 = const2 + row(2 * p)
        const9 = row(64 + 9 * 7)
        for p in range(8, 11):
            const9 = const9 + row(64 + 9 * p)

        t9b = n * (T9_ROWS * EMB) + lane
        for dg in range(9):
            plsc.store_scatter(t9_v, [t9b + dg * EMB], row(64 + dg) + const9)
        for r in range(9, T9_ROWS):
            plsc.store_scatter(t9_v, [t9b + r * EMB], row(64 + r))

        qb = n * (NQ * 16 * EMB) + lane
        for j in range(NQ):
            r = [row(8 * j + k) for k in range(8)]
            u01 = [r[q & 1] + r[2 + (q >> 1)] for q in range(4)]
            if j == 0:
                u01 = [u + const2 for u in u01]
            u23 = [r[4 + (q & 1)] + r[6 + (q >> 1)] for q in range(4)]
            for q in range(16):
                plsc.store_scatter(
                    quad_v,
                    [qb + (j * 16 + q) * EMB],
                    u01[q & 3] + u23[q >> 2],
                )

    def build_pair(i, carry):
        for b, stage, sem in ((0, stage0, s0), (1, stage1, s1)):
            n = 2 * i + b
            pltpu.make_async_copy(
                table_hbm.at[pl.ds(0, SLICE)], stage, sem).wait()
            build_feature(n, stage)

            @pl.when(i < NFEAT // 2 - 1)
            def _():
                nn = 2 * (i + 1) + b
                pltpu.async_copy(
                    table_hbm.at[pl.ds(nn * SLICE, SLICE)], stage, sem)
        return carry

    lax.fori_loop(0, NFEAT // 2, build_pair, 0)
    hx.wait()

    ninth = jnp.float32(1.0 / 9.0)

    def strip(n, chunk, buf):
        off = n * BPW + chunk * CHUNK
        xf = x_v[pl.ds(off, L)]
        x0 = (xf * MULT).astype(jnp.int32)
        out_base = lane * OUTW + n * (2 * EMB)

        accs = None
        for j in range(NQ):
            q = x0 if j == 0 else lax.shift_right_logical(
                x0, jnp.full((L,), 4 * j, jnp.int32))
            q = q & jnp.full((L,), 15, jnp.int32)
            flat = n * (NQ * 16 * EMB) + (j * 16 + q) * EMB
            g = [plsc.load_gather(quad_v, [flat + d]) for d in range(EMB)]
            accs = g if accs is None else [a + b for a, b in zip(accs, g)]
        for d in range(EMB):
            plsc.store_scatter(buf, [out_base + d], accs[d])

        x = x0
        accs = None
        for p in range(NP9):
            quot = (x.astype(jnp.float32) * ninth).astype(jnp.int32)
            dig = x - quot * 9
            flat = (n * T9_ROWS + 9 * p + dig) * EMB
            g = [plsc.load_gather(t9_v, [flat + d]) for d in range(EMB)]
            accs = g if accs is None else [a + b for a, b in zip(accs, g)]
            x = quot
        for d in range(EMB):
            plsc.store_scatter(buf, [out_base + EMB + d], accs[d])

    handles = [None, None]
    for chunk in range(NCHUNK):
        b = chunk % 2
        buf, sem = (out0, so0) if b == 0 else (out1, so1)
        if handles[b] is not None:
            handles[b].wait()

        def n_body(n, carry, chunk=chunk, buf=buf):
            strip(n, chunk, buf)
            return carry

        lax.fori_loop(0, NFEAT, n_body, 0)
        dst = (wid * BPW + chunk * CHUNK) * OUTW
        handles[b] = pltpu.async_copy(
            buf, out_hbm.at[pl.ds(dst, CHUNK * OUTW)], sem)
    handles[0].wait()
    handles[1].wait()


_sc_kernel = functools.partial(
    pl.kernel,
    out_type=jax.ShapeDtypeStruct((BATCH * OUTW,), jnp.float32),
    mesh=plsc.VectorSubcoreMesh(core_axis_name="c", subcore_axis_name="s"),
    compiler_params=pltpu.CompilerParams(needs_layout_passes=False),
    scratch_types=[
        pltpu.VMEM((NFEAT * BPW,), jnp.float32),
        pltpu.VMEM((SLICE,), jnp.float32),
        pltpu.VMEM((SLICE,), jnp.float32),
        pltpu.VMEM((QUAD_WORDS,), jnp.float32),
        pltpu.VMEM((T9_WORDS,), jnp.float32),
        pltpu.VMEM((CHUNK * OUTW,), jnp.float32),
        pltpu.VMEM((CHUNK * OUTW,), jnp.float32),
        pltpu.SemaphoreType.DMA,
        pltpu.SemaphoreType.DMA,
        pltpu.SemaphoreType.DMA,
        pltpu.SemaphoreType.DMA,
        pltpu.SemaphoreType.DMA,
    ],
)(_sc_body)


@jax.jit
def kernel(inputs, embedding_table):
    x_tiled = inputs.reshape(NW, BPW, NFEAT).transpose(0, 2, 1).reshape(-1)
    out = _sc_kernel(x_tiled, embedding_table.reshape(-1))
    return out.reshape(BATCH, OUTW)


# speedup vs baseline: 1.5051x; 1.5051x over previous
import functools
import math

import jax
import jax.numpy as jnp
from jax import lax
from jax.experimental import pallas as pl
from jax.experimental.pallas import tpu as pltpu
from jax.experimental.pallas import tpu_sc as plsc

EMB = 16
NFEAT = 26
BATCH = 4096
VOCAB = 163
MULT = 1000000.0

NC, NS, L = 2, 16, 16
NW = NC * NS
BPW = BATCH // NW
CHUNK = 16
NCHUNK = BPW // CHUNK
OUTW = NFEAT * 2 * EMB
SLICE = VOCAB * EMB
NQ = 5
NP9 = 7
QGRP = 272
QUAD_WORDS = NFEAT * NQ * QGRP
T9GRP = 144
T9_WORDS = NFEAT * NP9 * T9GRP


def _sc_body(x_hbm, table_hbm, out_hbm,
             x_v, stage0, stage1, quad_v, t9_v, out0, out1,
             sx, s0, s1, so0, so1):
    wid = lax.axis_index("s") * NC + lax.axis_index("c")

    hx = pltpu.async_copy(
        x_hbm.at[pl.ds(wid * (NFEAT * BPW), NFEAT * BPW)], x_v, sx)
    h0 = pltpu.async_copy(table_hbm.at[pl.ds(0, SLICE)], stage0, s0)
    h1 = pltpu.async_copy(table_hbm.at[pl.ds(SLICE, SLICE)], stage1, s1)

    lane = lax.iota(jnp.int32, L)
    lane9 = lane * 9
    lane17 = lane * 17

    def build_feature(n, stage):
        def row(r):
            return plsc.load_gather(stage, [r * EMB + lane])

        const2 = row(2 * 20)
        for p in range(21, 32):
            const2 = const2 + row(2 * p)
        const9 = row(64 + 9 * 7)
        for p in range(8, 11):
            const9 = const9 + row(64 + 9 * p)

        for p in range(NP9):
            t9b = (n * NP9 + p) * T9GRP + lane9
            for dg in range(9):
                v = row(64 + 9 * p + dg)
                if p == 0:
                    v = v + const9
                plsc.store_scatter(t9_v, [t9b + dg], v)

        for j in range(NQ):
            qb = (n * NQ + j) * QGRP + lane17
            r = [row(8 * j + k) for k in range(8)]
            u01 = [r[q & 1] + r[2 + (q >> 1)] for q in range(4)]
            if j == 0:
                u01 = [u + const2 for u in u01]
            u23 = [r[4 + (q & 1)] + r[6 + (q >> 1)] for q in range(4)]
            for q in range(16):
                plsc.store_scatter(
                    quad_v, [qb + q], u01[q & 3] + u23[q >> 2])

    def build_pair(i, carry):
        for b, stage, sem in ((0, stage0, s0), (1, stage1, s1)):
            n = 2 * i + b
            pltpu.make_async_copy(
                table_hbm.at[pl.ds(0, SLICE)], stage, sem).wait()
            build_feature(n, stage)

            @pl.when(i < NFEAT // 2 - 1)
            def _():
                nn = 2 * (i + 1) + b
                pltpu.async_copy(
                    table_hbm.at[pl.ds(nn * SLICE, SLICE)], stage, sem)
        return carry

    lax.fori_loop(0, NFEAT // 2, build_pair, 0)
    hx.wait()

    ninth = jnp.float32(1.0 / 9.0)

    def strip(n, chunk, buf):
        off = n * BPW + chunk * CHUNK
        xf = x_v[pl.ds(off, L)]
        x0 = (xf * MULT).astype(jnp.int32)
        out_base = lane * OUTW + n * (2 * EMB)

        accs = None
        for j in range(NQ):
            q = x0 if j == 0 else lax.shift_right_logical(
                x0, jnp.full((L,), 4 * j, jnp.int32))
            q = q & jnp.full((L,), 15, jnp.int32)
            flat = (n * NQ + j) * QGRP + q
            g = [plsc.load_gather(quad_v, [flat + 17 * d])
                 for d in range(EMB)]
            accs = g if accs is None else [a + b for a, b in zip(accs, g)]
        for d in range(EMB):
            plsc.store_scatter(buf, [out_base + d], accs[d])

        x = x0
        accs = None
        for p in range(NP9):
            quot = (x.astype(jnp.float32) * ninth).astype(jnp.int32)
            dig = x - quot * 9
            flat = (n * NP9 + p) * T9GRP + dig
            g = [plsc.load_gather(t9_v, [flat + 9 * d]) for d in range(EMB)]
            accs = g if accs is None else [a + b for a, b in zip(accs, g)]
            x = quot
        for d in range(EMB):
            plsc.store_scatter(buf, [out_base + EMB + d], accs[d])

    handles = [None, None]
    for chunk in range(NCHUNK):
        b = chunk % 2
        buf, sem = (out0, so0) if b == 0 else (out1, so1)
        if handles[b] is not None:
            handles[b].wait()

        def n_body(n, carry, chunk=chunk, buf=buf):
            strip(n, chunk, buf)
            return carry

        lax.fori_loop(0, NFEAT, n_body, 0)
        dst = (wid * BPW + chunk * CHUNK) * OUTW
        handles[b] = pltpu.async_copy(
            buf, out_hbm.at[pl.ds(dst, CHUNK * OUTW)], sem)
    handles[0].wait()
    handles[1].wait()


_sc_kernel = functools.partial(
    pl.kernel,
    out_type=jax.ShapeDtypeStruct((BATCH * OUTW,), jnp.float32),
    mesh=plsc.VectorSubcoreMesh(core_axis_name="c", subcore_axis_name="s"),
    compiler_params=pltpu.CompilerParams(needs_layout_passes=False),
    scratch_types=[
        pltpu.VMEM((NFEAT * BPW,), jnp.float32),
        pltpu.VMEM((SLICE,), jnp.float32),
        pltpu.VMEM((SLICE,), jnp.float32),
        pltpu.VMEM((QUAD_WORDS,), jnp.float32),
        pltpu.VMEM((T9_WORDS,), jnp.float32),
        pltpu.VMEM((CHUNK * OUTW,), jnp.float32),
        pltpu.VMEM((CHUNK * OUTW,), jnp.float32),
        pltpu.SemaphoreType.DMA,
        pltpu.SemaphoreType.DMA,
        pltpu.SemaphoreType.DMA,
        pltpu.SemaphoreType.DMA,
        pltpu.SemaphoreType.DMA,
    ],
)(_sc_body)


@jax.jit
def kernel(inputs, embedding_table):
    x_tiled = inputs.reshape(NW, BPW, NFEAT).transpose(0, 2, 1).reshape(-1)
    out = _sc_kernel(x_tiled, embedding_table.reshape(-1))
    return out.reshape(BATCH, OUTW)


# speedup vs baseline: 1.6711x; 1.1103x over previous
import functools
import math

import jax
import jax.numpy as jnp
from jax import lax
from jax.experimental import pallas as pl
from jax.experimental.pallas import tpu as pltpu
from jax.experimental.pallas import tpu_sc as plsc

EMB = 16
NFEAT = 26
BATCH = 4096
VOCAB = 163
MULT = 1000000.0

NC, NS, L = 2, 16, 16
NW = NC * NS
BPW = BATCH // NW
CHUNK = 16
NCHUNK = BPW // CHUNK
OUTW = NFEAT * 2 * EMB
SLICE = VOCAB * EMB
NQ = 5
NP9 = 7
QGRP = 272
QUAD_WORDS = NFEAT * NQ * QGRP
T9GRP = 144
T9_WORDS = NFEAT * NP9 * T9GRP


def _sc_body(x_hbm, table_hbm, out_hbm,
             x_v, stage0, stage1, quad_v, t9_v, out0, out1, tmp_v,
             sx, s0, s1, so0, so1):
    wid = lax.axis_index("s") * NC + lax.axis_index("c")

    hx = pltpu.async_copy(
        x_hbm.at[pl.ds(wid * (NFEAT * BPW), NFEAT * BPW)], x_v, sx)
    h0 = pltpu.async_copy(table_hbm.at[pl.ds(0, SLICE)], stage0, s0)
    h1 = pltpu.async_copy(table_hbm.at[pl.ds(SLICE, SLICE)], stage1, s1)

    lane = lax.iota(jnp.int32, L)
    lane9 = lane * 9
    lane17 = lane * 17
    lane16 = lane * 16
    lane26 = lane * NFEAT
    xl = [(lane + i) & 15 for i in range(L)]

    def build_feature(n, stage):
        def row(r):
            return plsc.load_gather(stage, [r * EMB + lane])

        const2 = row(2 * 20)
        for p in range(21, 32):
            const2 = const2 + row(2 * p)
        const9 = row(64 + 9 * 7)
        for p in range(8, 11):
            const9 = const9 + row(64 + 9 * p)

        for p in range(NP9):
            t9b = (n * NP9 + p) * T9GRP + lane9
            for dg in range(9):
                v = row(64 + 9 * p + dg)
                if p == 0:
                    v = v + const9
                plsc.store_scatter(t9_v, [t9b + dg], v)

        for j in range(NQ):
            qb = (n * NQ + j) * QGRP + lane17
            r = [row(8 * j + k) for k in range(8)]
            u01 = [r[q & 1] + r[2 + (q >> 1)] for q in range(4)]
            if j == 0:
                u01 = [u + const2 for u in u01]
            u23 = [r[4 + (q & 1)] + r[6 + (q >> 1)] for q in range(4)]
            for q in range(16):
                plsc.store_scatter(
                    quad_v, [qb + q], u01[q & 3] + u23[q >> 2])

    def build_pair(i, carry):
        for b, stage, sem in ((0, stage0, s0), (1, stage1, s1)):
            n = 2 * i + b
            pltpu.make_async_copy(
                table_hbm.at[pl.ds(0, SLICE)], stage, sem).wait()
            build_feature(n, stage)

            @pl.when(i < NFEAT // 2 - 1)
            def _():
                nn = 2 * (i + 1) + b
                pltpu.async_copy(
                    table_hbm.at[pl.ds(nn * SLICE, SLICE)], stage, sem)
        return carry

    lax.fori_loop(0, NFEAT // 2, build_pair, 0)
    hx.wait()

    ninth = jnp.float32(1.0 / 9.0)

    def strip(n, chunk, buf):
        xf = plsc.load_gather(x_v, [chunk * (CHUNK * NFEAT) + lane26 + n])
        x0 = (xf * MULT).astype(jnp.int32)

        def flush(accs, colbase):
            for d in range(EMB):
                plsc.store_scatter(tmp_v, [xl[d] + d * 16], accs[d])
            for r in range(L):
                t = plsc.load_gather(tmp_v, [lane16 + xl[r]])
                buf[pl.ds(r * OUTW + colbase, EMB)] = t

        accs = None
        for j in range(NQ):
            q = x0 if j == 0 else lax.shift_right_logical(
                x0, jnp.full((L,), 4 * j, jnp.int32))
            q = q & jnp.full((L,), 15, jnp.int32)
            flat = (n * NQ + j) * QGRP + q
            g = [plsc.load_gather(quad_v, [flat + 17 * d])
                 for d in range(EMB)]
            accs = g if accs is None else [a + b for a, b in zip(accs, g)]
        flush(accs, n * (2 * EMB))

        x = x0
        accs = None
        for p in range(NP9):
            quot = (x.astype(jnp.float32) * ninth).astype(jnp.int32)
            dig = x - quot * 9
            flat = (n * NP9 + p) * T9GRP + dig
            g = [plsc.load_gather(t9_v, [flat + 9 * d]) for d in range(EMB)]
            accs = g if accs is None else [a + b for a, b in zip(accs, g)]
            x = quot
        flush(accs, n * (2 * EMB) + EMB)

    handles = [None, None]
    for chunk in range(NCHUNK):
        b = chunk % 2
        buf, sem = (out0, so0) if b == 0 else (out1, so1)
        if handles[b] is not None:
            handles[b].wait()

        def n_body(n, carry, chunk=chunk, buf=buf):
            strip(n, chunk, buf)
            return carry

        lax.fori_loop(0, NFEAT, n_body, 0)
        dst = (wid * BPW + chunk * CHUNK) * OUTW
        handles[b] = pltpu.async_copy(
            buf, out_hbm.at[pl.ds(dst, CHUNK * OUTW)], sem)
    handles[0].wait()
    handles[1].wait()


_sc_kernel = functools.partial(
    pl.kernel,
    out_type=jax.ShapeDtypeStruct((BATCH * OUTW,), jnp.float32),
    mesh=plsc.VectorSubcoreMesh(core_axis_name="c", subcore_axis_name="s"),
    compiler_params=pltpu.CompilerParams(needs_layout_passes=False),
    scratch_types=[
        pltpu.VMEM((NFEAT * BPW,), jnp.float32),
        pltpu.VMEM((SLICE,), jnp.float32),
        pltpu.VMEM((SLICE,), jnp.float32),
        pltpu.VMEM((QUAD_WORDS,), jnp.float32),
        pltpu.VMEM((T9_WORDS,), jnp.float32),
        pltpu.VMEM((CHUNK * OUTW,), jnp.float32),
        pltpu.VMEM((CHUNK * OUTW,), jnp.float32),
        pltpu.VMEM((L * L,), jnp.float32),
        pltpu.SemaphoreType.DMA,
        pltpu.SemaphoreType.DMA,
        pltpu.SemaphoreType.DMA,
        pltpu.SemaphoreType.DMA,
        pltpu.SemaphoreType.DMA,
    ],
)(_sc_body)


@jax.jit
def kernel(inputs, embedding_table):
    out = _sc_kernel(inputs.reshape(-1), embedding_table.reshape(-1))
    return out.reshape(BATCH, OUTW)
